# baseline (speedup 1.0000x reference)
import functools

import jax
import jax.numpy as jnp
from jax import lax
from jax.experimental import pallas as pl
from jax.experimental.pallas import tpu as pltpu
from jax.experimental.pallas import tpu_sc as plsc

PH, PW, KK = 16, 16, 2
B, C, H, W = 8, 96, 224, 224
N = PH * PW * KK
NPTS = B * N
HW = H * W
CHW = C * HW
NWORKERS = 32
NIDX = 4 * C * 16
BB = 2
NPTS_CALL = BB * N
PTS_PER_WORKER = NPTS_CALL // NWORKERS
GROUPS = PTS_PER_WORKER // 16


def _sc_sample(gx, gy, xflat):
    mesh = plsc.VectorSubcoreMesh(core_axis_name="c", subcore_axis_name="s")

    @functools.partial(
        pl.kernel,
        mesh=mesh,
        compiler_params=pltpu.CompilerParams(needs_layout_passes=False),
        out_type=jax.ShapeDtypeStruct((NPTS_CALL * C,), jnp.float32),
        scratch_types=[
            pltpu.VMEM((PTS_PER_WORKER,), jnp.float32),
            pltpu.VMEM((PTS_PER_WORKER,), jnp.float32),
            pltpu.VMEM((NIDX,), jnp.int32),
            pltpu.VMEM((NIDX,), jnp.int32),
            pltpu.VMEM((NIDX,), jnp.float32),
            pltpu.VMEM((NIDX,), jnp.float32),
            pltpu.VMEM((16 * C,), jnp.float32),
            pltpu.VMEM((16 * C,), jnp.float32),
            pltpu.VMEM((16 * C,), jnp.float32),
            pltpu.SemaphoreType.DMA,
            pltpu.SemaphoreType.DMA,
        ],
    )
    def body(gx_hbm, gy_hbm, x_hbm, pf_hbm, cx_v, cy_v, idx0_v, idx1_v,
             dat0_v, dat1_v, obuf0_v, obuf1_v, tbuf_v, sem0, sem1):
        wid = lax.axis_index("s") * 2 + lax.axis_index("c")
        base = wid * PTS_PER_WORKER
        pltpu.sync_copy(gx_hbm.at[pl.ds(base, PTS_PER_WORKER)], cx_v)
        pltpu.sync_copy(gy_hbm.at[pl.ds(base, PTS_PER_WORKER)], cy_v)
        b_off = (wid // (NWORKERS // BB)) * CHW
        piota = lax.iota(jnp.int32, 16)
        tidx = [(piota + 16 * j) * 16 for j in range(C // 16)]

        def build(off, idx_v):
            vx = cx_v[pl.ds(off, 16)]
            vy = cy_v[pl.ds(off, 16)]
            ix = (vx + 1.0) * (W / 2.0) - 0.5
            iy = (vy + 1.0) * (H / 2.0) - 0.5
            tx = ix.astype(jnp.int32)
            ty = iy.astype(jnp.int32)
            ix0 = jnp.where(ix < tx.astype(jnp.float32), tx - 1, tx)
            iy0 = jnp.where(iy < ty.astype(jnp.float32), ty - 1, ty)
            fx1 = ix - ix0.astype(jnp.float32)
            fy1 = iy - iy0.astype(jnp.float32)
            fx0 = 1.0 - fx1
            fy0 = 1.0 - fy1
            ix1 = ix0 + 1
            iy1 = iy0 + 1
            zero = jnp.zeros((16,), jnp.float32)
            wx0 = jnp.where(ix0 >= 0, fx0, zero)
            wx1 = jnp.where(ix1 <= W - 1, fx1, zero)
            wy0 = jnp.where(iy0 >= 0, fy0, zero)
            wy1 = jnp.where(iy1 <= H - 1, fy1, zero)
            w00 = wx0 * wy0
            w01 = wx1 * wy0
            w10 = wx0 * wy1
            w11 = wx1 * wy1
            x0c = jnp.maximum(ix0, 0)
            x1c = jnp.minimum(ix1, W - 1)
            y0c = jnp.maximum(iy0, 0)
            y1c = jnp.minimum(iy1, H - 1)
            o00 = b_off + y0c * W + x0c
            o01 = b_off + y0c * W + x1c
            o10 = b_off + y1c * W + x0c
            o11 = b_off + y1c * W + x1c
            obase = (o00, o01, o10, o11)
            for k in range(4):
                ok = obase[k]
                for c in range(C):
                    idx_v[pl.ds((k * C + c) * 16, 16)] = ok + c * HW
            return (w00, w01, w10, w11)

        def accum(ws4, dat_v, obuf_v, off):
            for c in range(C):
                acc = dat_v[pl.ds((0 * C + c) * 16, 16)] * ws4[0]
                acc = acc + dat_v[pl.ds((1 * C + c) * 16, 16)] * ws4[1]
                acc = acc + dat_v[pl.ds((2 * C + c) * 16, 16)] * ws4[2]
                acc = acc + dat_v[pl.ds((3 * C + c) * 16, 16)] * ws4[3]
                tbuf_v[pl.ds(c * 16, 16)] = acc
            for p in range(16):
                for j in range(C // 16):
                    ov = plsc.load_gather(tbuf_v, [tidx[j] + p])
                    obuf_v[pl.ds(p * C + j * 16, 16)] = ov
            pltpu.sync_copy(obuf_v, pf_hbm.at[pl.ds((base + off) * C, 16 * C)])

        def pair(i, carry):
            off0 = pl.multiple_of(i * 32, 32)
            off1 = off0 + 16
            ws0 = build(off0, idx0_v)
            cp0 = pltpu.async_copy(x_hbm.at[idx0_v], dat0_v, sem0)
            ws1 = build(off1, idx1_v)
            cp1 = pltpu.async_copy(x_hbm.at[idx1_v], dat1_v, sem1)
            cp0.wait()
            accum(ws0, dat0_v, obuf0_v, off0)
            cp1.wait()
            accum(ws1, dat1_v, obuf1_v, off1)
            return carry

        lax.fori_loop(0, GROUPS // 2, pair, 0)

    return body(gx, gy, xflat)


def kernel(x):
    x = lax.stop_gradient(x)
    block_size = 2.0 / PH
    key = jax.random.key(1)
    block_coords = jax.random.uniform(key, (B, PH, PW, KK, 2), dtype=x.dtype) * block_size
    hs, ws = jnp.meshgrid(jnp.linspace(-1.0, 1.0 - block_size, PH),
                          jnp.linspace(-1.0, 1.0 - block_size, PW), indexing="ij")
    hs = hs.reshape(1, PH, PW, 1)
    ws = ws.reshape(1, PH, PW, 1)
    c0 = block_coords[..., 0] + hs
    c1 = block_coords[..., 1] + ws
    coords = jnp.stack([c0, c1], axis=-1).reshape(B, N, 2)
    gx = coords[..., 0].reshape(-1)
    gy = coords[..., 1].reshape(-1)
    parts = []
    for b0 in range(0, B, BB):
        xflat = x[b0:b0 + BB].reshape(-1)
        sl = slice(b0 * N, (b0 + BB) * N)
        parts.append(_sc_sample(gx[sl], gy[sl], xflat))
    pf = jnp.concatenate(parts)
    return coords, pf.reshape(B, N, C)


# speedup vs baseline: 1.2362x; 1.2362x over previous
import functools

import jax
import jax.numpy as jnp
from jax import lax
from jax.experimental import pallas as pl
from jax.experimental.pallas import tpu as pltpu
from jax.experimental.pallas import tpu_sc as plsc

PH, PW, KK = 16, 16, 2
B, C, H, W = 8, 96, 224, 224
N = PH * PW * KK
NPTS = B * N
HW = H * W
CHW = C * HW
NWORKERS = 32
NIDX = 4 * C * 16
BB = 2
NPTS_CALL = BB * N
PTS_PER_WORKER = NPTS_CALL // NWORKERS
GROUPS = PTS_PER_WORKER // 16
HWP = 448 * 128


def _relayout_body(i_ref, o_ref):
    blk = i_ref.shape[0]
    v = i_ref[...].reshape(blk, 28, 8, 224)
    o_ref[:, :, 0, :, :] = v[:, :, :, :128]
    o_ref[:, :, 1, :, :96] = v[:, :, :, 128:]


def _tc_relayout(x3):
    n = x3.shape[0]
    blk = 8
    return pl.pallas_call(
        _relayout_body,
        grid=(n // blk,),
        in_specs=[pl.BlockSpec((blk, 224, 224), lambda i: (i, 0, 0))],
        out_specs=pl.BlockSpec((blk, 28, 2, 8, 128), lambda i: (i, 0, 0, 0, 0)),
        out_shape=jax.ShapeDtypeStruct((n, 28, 2, 8, 128), jnp.float32),
    )(x3)


def _sc_sample(gx, gy, xflat):
    mesh = plsc.VectorSubcoreMesh(core_axis_name="c", subcore_axis_name="s")

    @functools.partial(
        pl.kernel,
        mesh=mesh,
        compiler_params=pltpu.CompilerParams(needs_layout_passes=False),
        out_type=jax.ShapeDtypeStruct((NPTS_CALL * C,), jnp.float32),
        scratch_types=[
            pltpu.VMEM((PTS_PER_WORKER,), jnp.float32),
            pltpu.VMEM((PTS_PER_WORKER,), jnp.float32),
            pltpu.VMEM((NIDX,), jnp.int32),
            pltpu.VMEM((NIDX,), jnp.int32),
            pltpu.VMEM((NIDX,), jnp.float32),
            pltpu.VMEM((NIDX,), jnp.float32),
            pltpu.VMEM((16 * C,), jnp.float32),
            pltpu.VMEM((16 * C,), jnp.float32),
            pltpu.VMEM((16 * C,), jnp.float32),
            pltpu.SemaphoreType.DMA,
            pltpu.SemaphoreType.DMA,
        ],
    )
    def body(gx_hbm, gy_hbm, x_hbm, pf_hbm, cx_v, cy_v, idx0_v, idx1_v,
             dat0_v, dat1_v, obuf0_v, obuf1_v, tbuf_v, sem0, sem1):
        wid = lax.axis_index("s") * 2 + lax.axis_index("c")
        base = wid * PTS_PER_WORKER
        pltpu.sync_copy(gx_hbm.at[pl.ds(base, PTS_PER_WORKER)], cx_v)
        pltpu.sync_copy(gy_hbm.at[pl.ds(base, PTS_PER_WORKER)], cy_v)
        b_off = (wid // (NWORKERS // BB)) * (C * HWP)
        piota = lax.iota(jnp.int32, 16)
        tidx = [(piota + 16 * j) * 16 for j in range(C // 16)]

        def paddr(y, x):
            return ((y >> 3) * 2048 + (y & 7) * 128
                    + (x >> 7) * 1024 + (x & 127))

        def build(off, idx_v):
            vx = cx_v[pl.ds(off, 16)]
            vy = cy_v[pl.ds(off, 16)]
            ix = (vx + 1.0) * (W / 2.0) - 0.5
            iy = (vy + 1.0) * (H / 2.0) - 0.5
            tx = ix.astype(jnp.int32)
            ty = iy.astype(jnp.int32)
            ix0 = jnp.where(ix < tx.astype(jnp.float32), tx - 1, tx)
            iy0 = jnp.where(iy < ty.astype(jnp.float32), ty - 1, ty)
            fx1 = ix - ix0.astype(jnp.float32)
            fy1 = iy - iy0.astype(jnp.float32)
            fx0 = 1.0 - fx1
            fy0 = 1.0 - fy1
            ix1 = ix0 + 1
            iy1 = iy0 + 1
            zero = jnp.zeros((16,), jnp.float32)
            wx0 = jnp.where(ix0 >= 0, fx0, zero)
            wx1 = jnp.where(ix1 <= W - 1, fx1, zero)
            wy0 = jnp.where(iy0 >= 0, fy0, zero)
            wy1 = jnp.where(iy1 <= H - 1, fy1, zero)
            w00 = wx0 * wy0
            w01 = wx1 * wy0
            w10 = wx0 * wy1
            w11 = wx1 * wy1
            x0c = jnp.maximum(ix0, 0)
            x1c = jnp.minimum(ix1, W - 1)
            y0c = jnp.maximum(iy0, 0)
            y1c = jnp.minimum(iy1, H - 1)
            o00 = b_off + paddr(y0c, x0c)
            o01 = b_off + paddr(y0c, x1c)
            o10 = b_off + paddr(y1c, x0c)
            o11 = b_off + paddr(y1c, x1c)
            obase = (o00, o01, o10, o11)
            for k in range(4):
                ok = obase[k]
                for c in range(C):
                    idx_v[pl.ds((k * C + c) * 16, 16)] = ok + c * HWP
            return (w00, w01, w10, w11)

        def accum(ws4, dat_v, obuf_v, off):
            for c in range(C):
                acc = dat_v[pl.ds((0 * C + c) * 16, 16)] * ws4[0]
                acc = acc + dat_v[pl.ds((1 * C + c) * 16, 16)] * ws4[1]
                acc = acc + dat_v[pl.ds((2 * C + c) * 16, 16)] * ws4[2]
                acc = acc + dat_v[pl.ds((3 * C + c) * 16, 16)] * ws4[3]
                tbuf_v[pl.ds(c * 16, 16)] = acc
            for p in range(16):
                for j in range(C // 16):
                    ov = plsc.load_gather(tbuf_v, [tidx[j] + p])
                    obuf_v[pl.ds(p * C + j * 16, 16)] = ov
            pltpu.sync_copy(obuf_v, pf_hbm.at[pl.ds((base + off) * C, 16 * C)])

        def pair(i, carry):
            off0 = pl.multiple_of(i * 32, 32)
            off1 = off0 + 16
            ws0 = build(off0, idx0_v)
            cp0 = pltpu.async_copy(x_hbm.at[idx0_v], dat0_v, sem0)
            ws1 = build(off1, idx1_v)
            cp1 = pltpu.async_copy(x_hbm.at[idx1_v], dat1_v, sem1)
            cp0.wait()
            accum(ws0, dat0_v, obuf0_v, off0)
            cp1.wait()
            accum(ws1, dat1_v, obuf1_v, off1)
            return carry

        lax.fori_loop(0, GROUPS // 2, pair, 0)

    return body(gx, gy, xflat)


def kernel(x):
    x = lax.stop_gradient(x)
    block_size = 2.0 / PH
    key = jax.random.key(1)
    block_coords = jax.random.uniform(key, (B, PH, PW, KK, 2), dtype=x.dtype) * block_size
    hs, ws = jnp.meshgrid(jnp.linspace(-1.0, 1.0 - block_size, PH),
                          jnp.linspace(-1.0, 1.0 - block_size, PW), indexing="ij")
    hs = hs.reshape(1, PH, PW, 1)
    ws = ws.reshape(1, PH, PW, 1)
    c0 = block_coords[..., 0] + hs
    c1 = block_coords[..., 1] + ws
    coords = jnp.stack([c0, c1], axis=-1).reshape(B, N, 2)
    gx = coords[..., 0].reshape(-1)
    gy = coords[..., 1].reshape(-1)
    parts = []
    x3 = x.reshape(B * C, H, W)
    for b0 in range(0, B, BB):
        xp = _tc_relayout(x3[b0 * C:(b0 + BB) * C])
        sl = slice(b0 * N, (b0 + BB) * N)
        parts.append(_sc_sample(gx[sl], gy[sl], xp.reshape(-1)))
    pf = jnp.concatenate(parts)
    return coords, pf.reshape(B, N, C)


# speedup vs baseline: 1.8420x; 1.4901x over previous
import functools

import jax
import jax.numpy as jnp
from jax import lax
from jax.experimental import pallas as pl
from jax.experimental.pallas import tpu as pltpu
from jax.experimental.pallas import tpu_sc as plsc

PH, PW, KK = 16, 16, 2
B, C, H, W = 8, 96, 224, 224
N = PH * PW * KK
NPTS = B * N
HW = H * W
CHW = C * HW
NWORKERS = 32
NIDX = 4 * C * 16
BB = 2
NPTS_CALL = BB * N
PTS_PER_WORKER = NPTS_CALL // NWORKERS
GROUPS = PTS_PER_WORKER // 16
HWP = 448 * 128


def _relayout_body(i_ref, o_ref):
    blk = i_ref.shape[0]
    v = i_ref[...].reshape(blk, 28, 8, 224)
    o_ref[:, :, 0, :, :] = v[:, :, :, :128]
    o_ref[:, :, 1, :, :96] = v[:, :, :, 128:]


def _tc_relayout(x3, base, n):
    blk = 8
    return pl.pallas_call(
        _relayout_body,
        grid=(n // blk,),
        in_specs=[pl.BlockSpec((blk, 224, 224),
                               lambda i: (base // blk + i, 0, 0))],
        out_specs=pl.BlockSpec((blk, 28, 2, 8, 128), lambda i: (i, 0, 0, 0, 0)),
        out_shape=jax.ShapeDtypeStruct((n, 28, 2, 8, 128), jnp.float32),
    )(x3)


def _sc_sample(gx, gy, xflat):
    mesh = plsc.VectorSubcoreMesh(core_axis_name="c", subcore_axis_name="s")

    @functools.partial(
        pl.kernel,
        mesh=mesh,
        compiler_params=pltpu.CompilerParams(needs_layout_passes=False),
        out_type=jax.ShapeDtypeStruct((NPTS_CALL * C,), jnp.float32),
        scratch_types=[
            pltpu.VMEM((PTS_PER_WORKER,), jnp.float32),
            pltpu.VMEM((PTS_PER_WORKER,), jnp.float32),
            pltpu.VMEM((NIDX,), jnp.int32),
            pltpu.VMEM((NIDX,), jnp.int32),
            pltpu.VMEM((NIDX,), jnp.float32),
            pltpu.VMEM((NIDX,), jnp.float32),
            pltpu.VMEM((16 * C,), jnp.float32),
            pltpu.VMEM((16 * C,), jnp.float32),
            pltpu.VMEM((16 * C,), jnp.float32),
            pltpu.SemaphoreType.DMA,
            pltpu.SemaphoreType.DMA,
        ],
    )
    def body(gx_hbm, gy_hbm, x_hbm, pf_hbm, cx_v, cy_v, idx0_v, idx1_v,
             dat0_v, dat1_v, obuf0_v, obuf1_v, tbuf_v, sem0, sem1):
        wid = lax.axis_index("s") * 2 + lax.axis_index("c")
        base = wid * PTS_PER_WORKER
        pltpu.sync_copy(gx_hbm.at[pl.ds(base, PTS_PER_WORKER)], cx_v)
        pltpu.sync_copy(gy_hbm.at[pl.ds(base, PTS_PER_WORKER)], cy_v)
        b_off = (wid // (NWORKERS // BB)) * (C * HWP)
        piota = lax.iota(jnp.int32, 16)
        tidx = [(piota + 16 * j) * 16 for j in range(C // 16)]

        def paddr(y, x):
            return ((y >> 3) * 2048 + (y & 7) * 128
                    + (x >> 7) * 1024 + (x & 127))

        def build(off, idx_v):
            vx = cx_v[pl.ds(off, 16)]
            vy = cy_v[pl.ds(off, 16)]
            ix = (vx + 1.0) * (W / 2.0) - 0.5
            iy = (vy + 1.0) * (H / 2.0) - 0.5
            tx = ix.astype(jnp.int32)
            ty = iy.astype(jnp.int32)
            ix0 = jnp.where(ix < tx.astype(jnp.float32), tx - 1, tx)
            iy0 = jnp.where(iy < ty.astype(jnp.float32), ty - 1, ty)
            fx1 = ix - ix0.astype(jnp.float32)
            fy1 = iy - iy0.astype(jnp.float32)
            fx0 = 1.0 - fx1
            fy0 = 1.0 - fy1
            ix1 = ix0 + 1
            iy1 = iy0 + 1
            zero = jnp.zeros((16,), jnp.float32)
            wx0 = jnp.where(ix0 >= 0, fx0, zero)
            wx1 = jnp.where(ix1 <= W - 1, fx1, zero)
            wy0 = jnp.where(iy0 >= 0, fy0, zero)
            wy1 = jnp.where(iy1 <= H - 1, fy1, zero)
            w00 = wx0 * wy0
            w01 = wx1 * wy0
            w10 = wx0 * wy1
            w11 = wx1 * wy1
            x0c = jnp.maximum(ix0, 0)
            x1c = jnp.minimum(ix1, W - 1)
            y0c = jnp.maximum(iy0, 0)
            y1c = jnp.minimum(iy1, H - 1)
            o00 = b_off + paddr(y0c, x0c)
            o01 = b_off + paddr(y0c, x1c)
            o10 = b_off + paddr(y1c, x0c)
            o11 = b_off + paddr(y1c, x1c)
            obase = (o00, o01, o10, o11)
            for k in range(4):
                ok = obase[k]
                for c in range(C):
                    idx_v[pl.ds((k * C + c) * 16, 16)] = ok + c * HWP
            return (w00, w01, w10, w11)

        def accum(ws4, dat_v, obuf_v, off):
            for c in range(C):
                acc = dat_v[pl.ds((0 * C + c) * 16, 16)] * ws4[0]
                acc = acc + dat_v[pl.ds((1 * C + c) * 16, 16)] * ws4[1]
                acc = acc + dat_v[pl.ds((2 * C + c) * 16, 16)] * ws4[2]
                acc = acc + dat_v[pl.ds((3 * C + c) * 16, 16)] * ws4[3]
                tbuf_v[pl.ds(c * 16, 16)] = acc
            for p in range(16):
                for j in range(C // 16):
                    ov = plsc.load_gather(tbuf_v, [tidx[j] + p])
                    obuf_v[pl.ds(p * C + j * 16, 16)] = ov
            pltpu.sync_copy(obuf_v, pf_hbm.at[pl.ds((base + off) * C, 16 * C)])

        def pair(i, carry):
            off0 = pl.multiple_of(i * 32, 32)
            off1 = off0 + 16
            ws0 = build(off0, idx0_v)
            cp0 = pltpu.async_copy(x_hbm.at[idx0_v], dat0_v, sem0)
            ws1 = build(off1, idx1_v)
            cp1 = pltpu.async_copy(x_hbm.at[idx1_v], dat1_v, sem1)
            cp0.wait()
            accum(ws0, dat0_v, obuf0_v, off0)
            cp1.wait()
            accum(ws1, dat1_v, obuf1_v, off1)
            return carry

        lax.fori_loop(0, GROUPS // 2, pair, 0)

    return body(gx, gy, xflat)


def kernel(x):
    x = lax.stop_gradient(x)
    block_size = 2.0 / PH
    key = jax.random.key(1)
    block_coords = jax.random.uniform(key, (B, PH, PW, KK, 2), dtype=x.dtype) * block_size
    hs, ws = jnp.meshgrid(jnp.linspace(-1.0, 1.0 - block_size, PH),
                          jnp.linspace(-1.0, 1.0 - block_size, PW), indexing="ij")
    hs = hs.reshape(1, PH, PW, 1)
    ws = ws.reshape(1, PH, PW, 1)
    c0 = block_coords[..., 0] + hs
    c1 = block_coords[..., 1] + ws
    coords = jnp.stack([c0, c1], axis=-1).reshape(B, N, 2)
    gx = coords[..., 0].reshape(-1)
    gy = coords[..., 1].reshape(-1)
    parts = []
    x3 = x.reshape(B * C, H, W)
    for b0 in range(0, B, BB):
        xp = _tc_relayout(x3, b0 * C, BB * C)
        sl = slice(b0 * N, (b0 + BB) * N)
        parts.append(_sc_sample(gx[sl], gy[sl], xp.reshape(-1)))
    pf = jnp.concatenate(parts)
    return coords, pf.reshape(B, N, C)
